# baseline (speedup 1.0000x reference)
import functools

import jax
import jax.numpy as jnp
from jax import lax
from jax.experimental import pallas as pl
from jax.experimental.pallas import tpu as pltpu
from jax.experimental.pallas import tpu_sc as plsc

_NC = 2
_NS = 16
_NW = _NC * _NS
_G = 128


_SPLIT = 503808


def _tr_body(a_ref, b_ref, out_ref):
    r = lax.broadcasted_iota(jnp.int32, (64, 64), 0)
    c = lax.broadcasted_iota(jnp.int32, (64, 64), 1)
    eye = (r == c).astype(jnp.float32)
    za = lax.dot_general(a_ref[...], eye, (((0,), (0,)), ((), ())),
                         preferred_element_type=jnp.float32)
    zb = lax.dot_general(b_ref[...], eye, (((0,), (0,)), ((), ())),
                         preferred_element_type=jnp.float32)
    out_ref[...] = jnp.concatenate([za, zb], axis=1)


def _tc_transpose(t64, tb=4096):
    grid = _SPLIT // tb
    return pl.pallas_call(
        _tr_body,
        grid=(grid,),
        in_specs=[
            pl.BlockSpec((64, tb), lambda i: (0, i)),
            pl.BlockSpec((64, tb), lambda i: (0, jnp.minimum(grid + i, 244))),
        ],
        out_specs=pl.BlockSpec((tb, 128), lambda i: (i, 0)),
        out_shape=jax.ShapeDtypeStruct((_SPLIT, 128), jnp.float32),
    )(t64, t64)


def _sc_gather(staged, idx_flat):
    n = idx_flat.shape[0]
    per_w = n // _NW
    chunks = per_w // _G
    mesh = plsc.VectorSubcoreMesh(core_axis_name="c", subcore_axis_name="s")

    @functools.partial(
        pl.kernel,
        mesh=mesh,
        out_type=jax.ShapeDtypeStruct((n, 128), jnp.float32),
        scratch_types=[
            pltpu.VMEM((per_w,), jnp.int32),
            pltpu.VMEM((_G, 128), jnp.float32),
            pltpu.SemaphoreType.DMA,
        ],
        compiler_params=pltpu.CompilerParams(use_tc_tiling_on_sc=False),
    )
    def gk(tab_hbm, idx_hbm, out_hbm, idx_v, rows_v, sem):
        wid = lax.axis_index("s") * _NC + lax.axis_index("c")
        base = wid * per_w
        pltpu.sync_copy(idx_hbm.at[pl.ds(base, per_w)], idx_v)

        def body(j, carry):
            off = j * _G
            pltpu.async_copy(
                tab_hbm.at[idx_v.at[pl.ds(off, _G)]], rows_v, sem
            ).wait()
            pltpu.sync_copy(rows_v, out_hbm.at[pl.ds(base + off, _G)])
            return carry

        lax.fori_loop(0, chunks, body, 0)

    return gk(staged, idx_flat)


def _tc_body(g_ref, h_ref, i_ref, o_ref, wi_ref, bi_ref, wo_ref, bo_ref,
             w_ref, bout_ref, out_ref):
    t1 = jnp.dot(i_ref[...], wi_ref[...], preferred_element_type=jnp.float32)
    t1 = t1 + bi_ref[...]
    o2 = o_ref[0]
    t2 = lax.dot_general(o2, wo_ref[...], (((0,), (0,)), ((), ())),
                         preferred_element_type=jnp.float32)
    t2 = t2 + bo_ref[...]
    g = jnp.where(h_ref[...] > 0, g_ref[:, 64:128], g_ref[:, 0:64])
    acc = jnp.dot(g, w_ref[0:64, :], preferred_element_type=jnp.float32)
    acc = acc + jnp.dot(t1, w_ref[64:96, :], preferred_element_type=jnp.float32)
    acc = acc + jnp.dot(t2, w_ref[96:128, :], preferred_element_type=jnp.float32)
    out_ref[...] = acc + bout_ref[...]


def _tc_dense_chunk(prev, gath, hsel, ing2, oth3, w_ing, b_ing, w_o, b_o,
                    w_out, b_out, s_off, s_cnt):
    l, _, b = oth3.shape
    n = b * l

    def body(*refs):
        if prev is not None:
            refs = refs[1:]
        _tc_body(*refs)

    specs = [
        pl.BlockSpec((b, 128), lambda s: (s, 0)),
        pl.BlockSpec((b, 1), lambda s: (s, 0)),
        pl.BlockSpec((b, 128), lambda s: (s_off + s, 0)),
        pl.BlockSpec((1, 64, b), lambda s: (s_off + s, 0, 0)),
        pl.BlockSpec((128, 32), lambda s: (0, 0)),
        pl.BlockSpec((1, 32), lambda s: (0, 0)),
        pl.BlockSpec((64, 32), lambda s: (0, 0)),
        pl.BlockSpec((1, 32), lambda s: (0, 0)),
        pl.BlockSpec((128, 128), lambda s: (0, 0)),
        pl.BlockSpec((1, 128), lambda s: (0, 0)),
    ]
    args = [gath, hsel, ing2, oth3, w_ing, b_ing.reshape(1, -1), w_o,
            b_o.reshape(1, -1), w_out, b_out.reshape(1, -1)]
    aliases = {}
    if prev is not None:
        specs = [pl.BlockSpec((b, 128), lambda s: (s_off + s, 0))] + specs
        args = [prev] + args
        aliases = {0: 0}

    return pl.pallas_call(
        body,
        grid=(s_cnt,),
        in_specs=specs,
        out_specs=pl.BlockSpec((b, 128), lambda s: (s_off + s, 0)),
        out_shape=jax.ShapeDtypeStruct((n, 128), jnp.float32),
        input_output_aliases=aliases,
        compiler_params=pltpu.CompilerParams(
            fuse_transposed_lhs_in_matmul=True),
    )(*args)


def kernel(recipe_id, ing, other_features, table, W_ing, b_ing, W_o, b_o,
           W_out, b_out):
    b, l = recipe_id.shape
    n = b * l
    idx = recipe_id.T.reshape(n)
    vidx = jnp.where(idx < _SPLIT, idx, idx - _SPLIT)
    hsel = (idx >= _SPLIT).astype(jnp.float32).reshape(n, 1)
    staged = _tc_transpose(table.T)
    ing2 = ing.transpose(1, 0, 2).reshape(n, 128)
    oth3 = other_features.transpose(1, 2, 0)
    nchunks = 2
    cs = l // nchunks
    out2 = None
    for k in range(nchunks):
        lo, hi = k * cs * b, (k + 1) * cs * b
        g = _sc_gather(staged, vidx[lo:hi])
        out2 = _tc_dense_chunk(out2, g, hsel[lo:hi], ing2, oth3, W_ing,
                               b_ing, W_o, b_o, W_out, b_out, k * cs, cs)
    return out2.reshape(l, b, 128).transpose(1, 0, 2)


# speedup vs baseline: 1.0861x; 1.0861x over previous
import functools

import jax
import jax.numpy as jnp
from jax import lax
from jax.experimental import pallas as pl
from jax.experimental.pallas import tpu as pltpu
from jax.experimental.pallas import tpu_sc as plsc

_NC = 2
_NS = 16
_NW = _NC * _NS
_G = 128


_SPLIT = 507904


def _tr_body(a_ref, b_ref, out_ref):
    r = lax.broadcasted_iota(jnp.int32, (64, 64), 0)
    c = lax.broadcasted_iota(jnp.int32, (64, 64), 1)
    eye = (r == c).astype(jnp.float32)
    za = lax.dot_general(a_ref[...], eye, (((0,), (0,)), ((), ())),
                         preferred_element_type=jnp.float32)
    zb = lax.dot_general(b_ref[...], eye, (((0,), (0,)), ((), ())),
                         preferred_element_type=jnp.float32)
    out_ref[...] = jnp.concatenate([za, zb], axis=1)


def _tc_transpose(t64, tb=16384):
    grid = _SPLIT // tb
    nb = pl.cdiv(t64.shape[1], tb) - 1
    return pl.pallas_call(
        _tr_body,
        grid=(grid,),
        in_specs=[
            pl.BlockSpec((64, tb), lambda i: (0, i)),
            pl.BlockSpec((64, tb), lambda i: (0, jnp.minimum(grid + i, nb))),
        ],
        out_specs=pl.BlockSpec((tb, 128), lambda i: (i, 0)),
        out_shape=jax.ShapeDtypeStruct((_SPLIT, 128), jnp.float32),
    )(t64, t64)


def _sc_gather(staged, idx_flat):
    n = idx_flat.shape[0]
    per_w = n // _NW
    chunks = per_w // _G
    mesh = plsc.VectorSubcoreMesh(core_axis_name="c", subcore_axis_name="s")

    @functools.partial(
        pl.kernel,
        mesh=mesh,
        out_type=jax.ShapeDtypeStruct((n, 128), jnp.float32),
        scratch_types=[
            pltpu.VMEM((per_w,), jnp.int32),
            pltpu.VMEM((_G, 128), jnp.float32),
            pltpu.SemaphoreType.DMA,
        ],
        compiler_params=pltpu.CompilerParams(use_tc_tiling_on_sc=False),
    )
    def gk(tab_hbm, idx_hbm, out_hbm, idx_v, rows_v, sem):
        wid = lax.axis_index("s") * _NC + lax.axis_index("c")
        base = wid * per_w
        pltpu.sync_copy(idx_hbm.at[pl.ds(base, per_w)], idx_v)

        def body(j, carry):
            off = j * _G
            pltpu.async_copy(
                tab_hbm.at[idx_v.at[pl.ds(off, _G)]], rows_v, sem
            ).wait()
            pltpu.sync_copy(rows_v, out_hbm.at[pl.ds(base + off, _G)])
            return carry

        lax.fori_loop(0, chunks, body, 0)

    return gk(staged, idx_flat)


def _tc_body(g_ref, h_ref, i_ref, o_ref, wi_ref, bi_ref, wo_ref, bo_ref,
             w_ref, bout_ref, out_ref):
    t1 = jnp.dot(i_ref[...], wi_ref[...], preferred_element_type=jnp.float32)
    t1 = t1 + bi_ref[...]
    o2 = o_ref[0]
    t2 = lax.dot_general(o2, wo_ref[...], (((0,), (0,)), ((), ())),
                         preferred_element_type=jnp.float32)
    t2 = t2 + bo_ref[...]
    g = jnp.where(h_ref[...] > 0, g_ref[:, 64:128], g_ref[:, 0:64])
    acc = jnp.dot(g, w_ref[0:64, :], preferred_element_type=jnp.float32)
    acc = acc + jnp.dot(t1, w_ref[64:96, :], preferred_element_type=jnp.float32)
    acc = acc + jnp.dot(t2, w_ref[96:128, :], preferred_element_type=jnp.float32)
    out_ref[...] = acc + bout_ref[...]


def _tc_dense_chunk(prev, gath, hsel, ing2, oth3, w_ing, b_ing, w_o, b_o,
                    w_out, b_out, s_off, s_cnt):
    l, _, b = oth3.shape
    n = b * l

    def body(*refs):
        if prev is not None:
            refs = refs[1:]
        _tc_body(*refs)

    specs = [
        pl.BlockSpec((b, 128), lambda s: (s, 0)),
        pl.BlockSpec((b, 1), lambda s: (s, 0)),
        pl.BlockSpec((b, 128), lambda s: (s_off + s, 0)),
        pl.BlockSpec((1, 64, b), lambda s: (s_off + s, 0, 0)),
        pl.BlockSpec((128, 32), lambda s: (0, 0)),
        pl.BlockSpec((1, 32), lambda s: (0, 0)),
        pl.BlockSpec((64, 32), lambda s: (0, 0)),
        pl.BlockSpec((1, 32), lambda s: (0, 0)),
        pl.BlockSpec((128, 128), lambda s: (0, 0)),
        pl.BlockSpec((1, 128), lambda s: (0, 0)),
    ]
    args = [gath, hsel, ing2, oth3, w_ing, b_ing.reshape(1, -1), w_o,
            b_o.reshape(1, -1), w_out, b_out.reshape(1, -1)]
    aliases = {}
    if prev is not None:
        specs = [pl.BlockSpec((b, 128), lambda s: (s_off + s, 0))] + specs
        args = [prev] + args
        aliases = {0: 0}

    return pl.pallas_call(
        body,
        grid=(s_cnt,),
        in_specs=specs,
        out_specs=pl.BlockSpec((b, 128), lambda s: (s_off + s, 0)),
        out_shape=jax.ShapeDtypeStruct((n, 128), jnp.float32),
        input_output_aliases=aliases,
        compiler_params=pltpu.CompilerParams(
            fuse_transposed_lhs_in_matmul=True),
    )(*args)


def kernel(recipe_id, ing, other_features, table, W_ing, b_ing, W_o, b_o,
           W_out, b_out):
    b, l = recipe_id.shape
    n = b * l
    idx = recipe_id.T.reshape(n)
    vidx = jnp.where(idx < _SPLIT, idx, idx - _SPLIT)
    hsel = (idx >= _SPLIT).astype(jnp.float32).reshape(n, 1)
    staged = _tc_transpose(table.T)
    ing2 = ing.transpose(1, 0, 2).reshape(n, 128)
    oth3 = other_features.transpose(1, 2, 0)
    nchunks = 2
    cs = l // nchunks
    out2 = None
    for k in range(nchunks):
        lo, hi = k * cs * b, (k + 1) * cs * b
        g = _sc_gather(staged, vidx[lo:hi])
        out2 = _tc_dense_chunk(out2, g, hsel[lo:hi], ing2, oth3, W_ing,
                               b_ing, W_o, b_o, W_out, b_out, k * cs, cs)
    return out2.reshape(l, b, 128).transpose(1, 0, 2)
